# Initial kernel scaffold; baseline (speedup 1.0000x reference)
import jax
import jax.numpy as jnp
from jax.experimental import pallas as pl


def kernel(feat0, feat1, feat2, W_shared, b_shared, W_gate, b_gate, W_exp, b_exp):
    raise NotImplementedError("write your pallas kernel here")


# speedup vs baseline: 2.0292x; 2.0292x over previous
import functools

import jax
import jax.numpy as jnp
from jax.experimental import pallas as pl
from jax.experimental.pallas import tpu as pltpu

B = 4096
D0, D1, D2 = 1024, 1024, 2048
F = 2048
E = 8

_INV_SQRT2 = 0.7071067811865476


def _gelu_exact(x):
    return x * (0.5 * (1.0 + jax.lax.erf(x * _INV_SQRT2)))


BM_A = 512
BN_A = 512


def _bf16_dot(a, b):
    return jax.lax.dot_general(
        a.astype(jnp.bfloat16), b.astype(jnp.bfloat16),
        (((1,), (0,)), ((), ())),
        preferred_element_type=jnp.float32)


def _shared_body(f0, f1, f2, w, b, h32, h16):
    acc = _bf16_dot(f0[...], w[0:D0, :])
    acc += _bf16_dot(f1[...], w[D0:D0 + D1, :])
    acc += _bf16_dot(f2[...], w[D0 + D1:D0 + D1 + D2, :])
    acc = acc + b[...]
    acc = _gelu_exact(acc)
    h32[...] = acc
    h16[...] = acc.astype(jnp.bfloat16)


def _shared_proj(f0, f1, f2, W_shared, b_shared):
    grid = (B // BM_A, F // BN_A)
    return pl.pallas_call(
        _shared_body,
        grid=grid,
        in_specs=[
            pl.BlockSpec((BM_A, D0), lambda i, j: (i, 0)),
            pl.BlockSpec((BM_A, D1), lambda i, j: (i, 0)),
            pl.BlockSpec((BM_A, D2), lambda i, j: (i, 0)),
            pl.BlockSpec((D0 + D1 + D2, BN_A), lambda i, j: (0, j)),
            pl.BlockSpec((1, BN_A), lambda i, j: (0, j)),
        ],
        out_specs=[
            pl.BlockSpec((BM_A, BN_A), lambda i, j: (i, j)),
            pl.BlockSpec((BM_A, BN_A), lambda i, j: (i, j)),
        ],
        out_shape=[
            jax.ShapeDtypeStruct((B, F), jnp.float32),
            jax.ShapeDtypeStruct((B, F), jnp.bfloat16),
        ],
        compiler_params=pltpu.CompilerParams(
            dimension_semantics=("parallel", "parallel")),
    )(f0, f1, f2, W_shared, b_shared.reshape(1, F))


BM_G = 1024


def _gate_body(h, wg, bg, wcomb):
    logits = _bf16_dot(h[...], wg[...]) + bg[...]
    m = jnp.max(logits, axis=1, keepdims=True)
    ex = jnp.exp(logits - m)
    p = ex / jnp.sum(ex, axis=1, keepdims=True)
    lane = jax.lax.broadcasted_iota(jnp.int32, p.shape, 1)
    i1 = jnp.argmax(p, axis=1)[:, None]
    w1 = jnp.max(p, axis=1, keepdims=True)
    p2 = jnp.where(lane == i1, -1.0, p)
    i2 = jnp.argmax(p2, axis=1)[:, None]
    w2 = jnp.max(p2, axis=1, keepdims=True)
    wcomb[...] = jnp.where(lane == i1, w1, 0.0) + jnp.where(lane == i2, w2, 0.0)


def _gating(h32, W_gate, b_gate):
    grid = (B // BM_G,)
    return pl.pallas_call(
        _gate_body,
        grid=grid,
        in_specs=[
            pl.BlockSpec((BM_G, F), lambda i: (i, 0)),
            pl.BlockSpec((F, E), lambda i: (0, 0)),
            pl.BlockSpec((1, E), lambda i: (0, 0)),
        ],
        out_specs=pl.BlockSpec((BM_G, E), lambda i: (i, 0)),
        out_shape=jax.ShapeDtypeStruct((B, E), jnp.float32),
        compiler_params=pltpu.CompilerParams(
            dimension_semantics=("parallel",)),
    )(h32, W_gate, b_gate.reshape(1, E))


BM_C = 1024


def _expert_body(h16, wexp, bexp, wc, out):
    e = pl.program_id(1)
    acc = jax.lax.dot_general(
        h16[...], wexp[0], (((1,), (0,)), ((), ())),
        preferred_element_type=jnp.float32) + bexp[0]
    eo = _gelu_exact(acc)
    lane = jax.lax.broadcasted_iota(jnp.int32, (BM_C, E), 1)
    w = jnp.sum(jnp.where(lane == e, wc[...], 0.0), axis=1, keepdims=True)
    contrib = w * eo

    @pl.when(e == 0)
    def _():
        out[...] = contrib

    @pl.when(e != 0)
    def _():
        out[...] += contrib


def _experts_dense(W16, b_exp, h16, wcomb):
    grid = (B // BM_C, E)
    return pl.pallas_call(
        _expert_body,
        grid=grid,
        in_specs=[
            pl.BlockSpec((BM_C, F), lambda i, e: (i, 0)),
            pl.BlockSpec((1, F, F), lambda i, e: (e, 0, 0)),
            pl.BlockSpec((1, 1, F), lambda i, e: (e, 0, 0)),
            pl.BlockSpec((BM_C, E), lambda i, e: (i, 0)),
        ],
        out_specs=pl.BlockSpec((BM_C, F), lambda i, e: (i, 0)),
        out_shape=jax.ShapeDtypeStruct((B, F), jnp.float32),
        compiler_params=pltpu.CompilerParams(
            dimension_semantics=("parallel", "arbitrary")),
    )(h16, W16, b_exp.reshape(E, 1, F), wcomb)


def kernel(feat0, feat1, feat2, W_shared, b_shared, W_gate, b_gate, W_exp, b_exp):
    W16 = W_exp.astype(jnp.bfloat16)
    h32, h16 = _shared_proj(feat0, feat1, feat2, W_shared, b_shared)
    wcomb = _gating(h32, W_gate, b_gate)
    return _experts_dense(W16, b_exp, h16, wcomb)
